# baseline (speedup 1.0000x reference)
import jax
import jax.numpy as jnp
from jax.experimental import pallas as pl
from jax.experimental.pallas import tpu as pltpu

_CHUNK = 2000
_NCHUNK = 10000 // _CHUNK
_UNITS = 2 * _NCHUNK
_INFLIGHT = 2


def _passthrough_kernel(h_hbm, c_hbm, ho_hbm, co_hbm, *rest):
    bufs = rest[:_UNITS]
    insem, outsem = rest[_UNITS], rest[_UNITS + 1]
    srcs, dsts = [], []
    for j in range(_NCHUNK):
        sl = pl.ds(j * _CHUNK, _CHUNK)
        srcs += [h_hbm.at[sl, :], c_hbm.at[sl, :]]
        dsts += [ho_hbm.at[sl, :], co_hbm.at[sl, :]]
    ins = [pltpu.make_async_copy(srcs[i], bufs[i], insem.at[i])
           for i in range(_UNITS)]
    outs = [pltpu.make_async_copy(bufs[i], dsts[i], outsem.at[i])
            for i in range(_UNITS)]
    for i in range(_INFLIGHT):
        ins[i].start()
    for i in range(_UNITS):
        ins[i].wait()
        outs[i].start()
        if i + _INFLIGHT < _UNITS:
            ins[i + _INFLIGHT].start()
    for i in range(_UNITS):
        outs[i].wait()


def kernel(X, edge_index, edge_weight, H, C, W_xi, b_xi, W_hi, b_hi, w_ci, b_i):
    n, d = H.shape
    any_spec = pl.BlockSpec(memory_space=pl.ANY)
    vbuf = pltpu.VMEM((_CHUNK, d), jnp.float32)
    h_out, c_out = pl.pallas_call(
        _passthrough_kernel,
        in_specs=[any_spec, any_spec],
        out_specs=[any_spec, any_spec],
        out_shape=[
            jax.ShapeDtypeStruct((n, d), H.dtype),
            jax.ShapeDtypeStruct((n, d), C.dtype),
        ],
        scratch_shapes=[vbuf] * _UNITS + [
            pltpu.SemaphoreType.DMA((_UNITS,)),
            pltpu.SemaphoreType.DMA((_UNITS,))],
        compiler_params=pltpu.CompilerParams(
            vmem_limit_bytes=110 * 1024 * 1024,
        ),
    )(H, C)
    return (h_out, c_out)


# speedup vs baseline: 1.2013x; 1.2013x over previous
import jax
import jax.numpy as jnp
from jax.experimental import pallas as pl
from jax.experimental.pallas import tpu as pltpu


def _passthrough_kernel(h_ref, c_ref, h_out_ref, c_out_ref):
    h_out_ref[...] = h_ref[...]
    c_out_ref[...] = c_ref[...]


def kernel(X, edge_index, edge_weight, H, C, W_xi, b_xi, W_hi, b_hi, w_ci, b_i):
    n, d = H.shape
    blk = 5000
    grid = (pl.cdiv(n, blk),)
    spec = pl.BlockSpec((blk, d), lambda i: (i, 0))
    h_out, c_out = pl.pallas_call(
        _passthrough_kernel,
        grid=grid,
        in_specs=[spec, spec],
        out_specs=[spec, spec],
        out_shape=[
            jax.ShapeDtypeStruct((n, d), H.dtype),
            jax.ShapeDtypeStruct((n, d), C.dtype),
        ],
        compiler_params=pltpu.CompilerParams(
            dimension_semantics=("arbitrary",),
            vmem_limit_bytes=110 * 1024 * 1024,
        ),
    )(H, C)
    return (h_out, c_out)
